# Initial kernel scaffold; baseline (speedup 1.0000x reference)
import jax
import jax.numpy as jnp
from jax.experimental import pallas as pl


def kernel(fake_validity_voxel_0, fake_validity_voxel_1, fake_validity_program, voxel_feature, att, mask, program_target_ratio, pooled_program_feature_from_voxel, cross_edge_voxel_index, cross_edge_program_index, program_class_cluster, max_out_program_index, area_index_in_voxel_feature):
    raise NotImplementedError("write your pallas kernel here")


# speedup vs baseline: 58.6032x; 58.6032x over previous
import functools

import jax
import jax.numpy as jnp
from jax import lax
from jax.experimental import pallas as pl
from jax.experimental.pallas import tpu as pltpu
from jax.experimental.pallas import tpu_sc as plsc

_NV = 100000
_E = 1600000
_NP = 2000
_NCLS = 6
_FDIM = 13
_NW = 32

_R1 = 3136
_NR1V = _R1 // 16
_NV_PAD = _NW * _R1
_VF_PAD = _NV_PAD * _FDIM

_EW = _E // _NW
_CH = 2000
_NCH = _EW // _CH
_NEV = _CH // 16

_mesh = plsc.VectorSubcoreMesh(core_axis_name="c", subcore_axis_name="s")
_sc_params = pltpu.CompilerParams(needs_layout_passes=False)


def _wid():
    return lax.axis_index("s") * 2 + lax.axis_index("c")


@functools.partial(
    pl.kernel,
    mesh=_mesh,
    out_type=jax.ShapeDtypeStruct((_NV_PAD,), jnp.float32),
    scratch_types=[
        pltpu.VMEM((_R1 * _FDIM,), jnp.float32),
        pltpu.VMEM((_R1,), jnp.float32),
        pltpu.VMEM((16,), jnp.int32),
    ],
    compiler_params=_sc_params,
)
def _col_extract(vf_hbm, col_hbm, out_hbm, slab, obuf, colv):
    wid = _wid()
    pltpu.sync_copy(col_hbm, colv)
    col = colv[...]
    pltpu.sync_copy(vf_hbm.at[pl.ds(wid * (_R1 * _FDIM), _R1 * _FDIM)], slab)

    def body(i, carry):
        idx = (lax.iota(jnp.int32, 16) + i * 16) * _FDIM + col
        obuf[pl.ds(i * 16, 16)] = plsc.load_gather(slab, [idx])
        return carry

    lax.fori_loop(0, _NR1V, body, 0)
    pltpu.sync_copy(obuf, out_hbm.at[pl.ds(wid * _R1, _R1)])


@functools.partial(
    pl.kernel,
    mesh=_mesh,
    out_type=jax.ShapeDtypeStruct((_NW, 96), jnp.float32),
    scratch_types=[
        pltpu.VMEM((_NV_PAD,), jnp.float32),
        pltpu.VMEM((_NP,), jnp.int32),
        pltpu.VMEM((_CH,), jnp.float32),
        pltpu.VMEM((_CH,), jnp.int32),
        pltpu.VMEM((_CH,), jnp.int32),
        pltpu.VMEM((96,), jnp.float32),
    ],
    compiler_params=_sc_params,
)
def _edge_accum(area_hbm, att_hbm, vox_hbm, prog_hbm, cls_hbm, out_hbm,
                area_t, cls_t, att_b, vox_b, prog_b, acc_b):
    wid = _wid()
    pltpu.sync_copy(area_hbm, area_t)
    pltpu.sync_copy(cls_hbm, cls_t)
    ebase = wid * _EW
    zero = jnp.zeros((16,), jnp.float32)

    def chunk_body(cidx, accs):
        base = ebase + cidx * _CH
        pltpu.sync_copy(att_hbm.at[pl.ds(base, _CH)], att_b)
        pltpu.sync_copy(vox_hbm.at[pl.ds(base, _CH)], vox_b)
        pltpu.sync_copy(prog_hbm.at[pl.ds(base, _CH)], prog_b)

        def vbody(v, accs):
            o = v * 16
            vox = vox_b[pl.ds(o, 16)]
            a = plsc.load_gather(area_t, [vox])
            val = att_b[pl.ds(o, 16)] * a
            prog = prog_b[pl.ds(o, 16)]
            cls = plsc.load_gather(cls_t, [prog])
            return tuple(
                acc + jnp.where(cls == c, val, 0.0)
                for c, acc in enumerate(accs))

        return lax.fori_loop(0, _NEV, vbody, accs)

    accs = lax.fori_loop(0, _NCH, chunk_body, (zero,) * _NCLS)
    for c in range(_NCLS):
        acc_b[pl.ds(c * 16, 16)] = accs[c]
    pltpu.sync_copy(acc_b, out_hbm.at[wid])


def _finish_body(part_ref, fv0_ref, fv1_ref, tgt_ref, tot_ref, adv_ref, tr_ref):
    adv = (-(jnp.sum(fv0_ref[...]) * (1.0 / 4096.0))
           - (jnp.sum(fv1_ref[...]) * (1.0 / 4096.0)))
    x = part_ref[...]
    s = [jnp.sum(x[:, c * 16:(c + 1) * 16]) for c in range(_NCLS)]
    tot_w = s[0] + s[1] + s[2] + s[3] + s[4] + s[5]
    inv = 1.0 / (tot_w + 1e-16)
    t = tgt_ref[...]
    losses = []
    for c in range(_NCLS):
        d = s[c] * inv - jnp.sum(t[:, c:c + 1])
        ad = jnp.abs(d)
        losses.append(jnp.where(ad < 1.0, 0.5 * d * d, ad - 0.5))
    tr = (losses[0] + losses[1] + losses[2] + losses[3] + losses[4]
          + losses[5]) * (1.0 / 6.0)
    tot = adv + tr
    tot_ref[...] = jnp.full((1, 1), tot, jnp.float32)
    adv_ref[...] = jnp.full((1, 1), adv, jnp.float32)
    tr_ref[...] = jnp.full((1, 1), tr, jnp.float32)


_finish = pl.pallas_call(
    _finish_body,
    out_shape=(jax.ShapeDtypeStruct((1, 1), jnp.float32),) * 3,
)


def kernel(fake_validity_voxel_0, fake_validity_voxel_1, fake_validity_program,
           voxel_feature, att, mask, program_target_ratio,
           pooled_program_feature_from_voxel, cross_edge_voxel_index,
           cross_edge_program_index, program_class_cluster,
           max_out_program_index, area_index_in_voxel_feature):
    vf_flat = jnp.pad(voxel_feature.reshape(-1), (0, _VF_PAD - _NV * _FDIM))
    colv = jnp.full((16,), area_index_in_voxel_feature, jnp.int32)
    area = _col_extract(vf_flat, colv)
    partials = _edge_accum(
        area,
        att.reshape(-1),
        cross_edge_voxel_index.astype(jnp.int32),
        cross_edge_program_index.astype(jnp.int32),
        program_class_cluster.astype(jnp.int32),
    )
    tot, adv, tr = _finish(
        partials,
        fake_validity_voxel_0.reshape(32, 128),
        fake_validity_voxel_1.reshape(32, 128),
        program_target_ratio.reshape(1, _NCLS),
    )
    total_loss = tot.reshape(())
    adversarial_loss = adv.reshape(())
    target_ratio_loss = tr.reshape(())
    link_prediction_loss = jnp.zeros(())
    return (total_loss, adversarial_loss, link_prediction_loss,
            target_ratio_loss)


# speedup vs baseline: 80.6788x; 1.3767x over previous
import functools

import jax
import jax.numpy as jnp
from jax import lax
from jax.experimental import pallas as pl
from jax.experimental.pallas import tpu as pltpu
from jax.experimental.pallas import tpu_sc as plsc

_NV = 100000
_E = 1600000
_NP = 2000
_NCLS = 6
_NW = 32

_EW = _E // _NW
_CH = 2000
_NCH = _EW // _CH
_NEV = _CH // 16

_mesh = plsc.VectorSubcoreMesh(core_axis_name="c", subcore_axis_name="s")
_sc_params = pltpu.CompilerParams(needs_layout_passes=False,
                                  use_tc_tiling_on_sc=False)


@functools.partial(
    pl.kernel,
    mesh=_mesh,
    out_type=jax.ShapeDtypeStruct((_NW, 96), jnp.float32),
    scratch_types=[
        pltpu.VMEM((_NV,), jnp.float32),
        pltpu.VMEM((_NP,), jnp.int32),
        pltpu.VMEM((_CH,), jnp.float32),
        pltpu.VMEM((_CH,), jnp.int32),
        pltpu.VMEM((_CH,), jnp.int32),
        pltpu.VMEM((96,), jnp.float32),
    ],
    compiler_params=_sc_params,
)
def _edge_accum(area_hbm, att_hbm, vox_hbm, prog_hbm, cls_hbm, out_hbm,
                area_t, cls_t, att_b, vox_b, prog_b, acc_b):
    wid = lax.axis_index("s") * 2 + lax.axis_index("c")
    pltpu.sync_copy(area_hbm, area_t)
    pltpu.sync_copy(cls_hbm, cls_t)
    ebase = wid * _EW
    zero = jnp.zeros((16,), jnp.float32)

    def chunk_body(cidx, accs):
        base = ebase + cidx * _CH
        pltpu.sync_copy(att_hbm.at[wid * _NCH + cidx], att_b)
        pltpu.sync_copy(vox_hbm.at[pl.ds(base, _CH)], vox_b)
        pltpu.sync_copy(prog_hbm.at[pl.ds(base, _CH)], prog_b)

        def vbody(v, accs):
            o = v * 16
            vox = vox_b[pl.ds(o, 16)]
            a = plsc.load_gather(area_t, [vox])
            val = att_b[pl.ds(o, 16)] * a
            prog = prog_b[pl.ds(o, 16)]
            cls = plsc.load_gather(cls_t, [prog])
            return tuple(
                acc + jnp.where(cls == c, val, 0.0)
                for c, acc in enumerate(accs))

        return lax.fori_loop(0, _NEV, vbody, accs)

    accs = lax.fori_loop(0, _NCH, chunk_body, (zero,) * _NCLS)
    for c in range(_NCLS):
        acc_b[pl.ds(c * 16, 16)] = accs[c]
    pltpu.sync_copy(acc_b, out_hbm.at[wid])


def _finish_body(part_ref, fv0_ref, fv1_ref, tgt_ref, tot_ref, adv_ref, tr_ref):
    adv = (-(jnp.sum(fv0_ref[...]) * (1.0 / 4096.0))
           - (jnp.sum(fv1_ref[...]) * (1.0 / 4096.0)))
    x = part_ref[...]
    s = [jnp.sum(x[:, c * 16:(c + 1) * 16]) for c in range(_NCLS)]
    tot_w = s[0] + s[1] + s[2] + s[3] + s[4] + s[5]
    inv = 1.0 / (tot_w + 1e-16)
    t = tgt_ref[...]
    losses = []
    for c in range(_NCLS):
        d = s[c] * inv - jnp.sum(t[:, c:c + 1])
        ad = jnp.abs(d)
        losses.append(jnp.where(ad < 1.0, 0.5 * d * d, ad - 0.5))
    tr = (losses[0] + losses[1] + losses[2] + losses[3] + losses[4]
          + losses[5]) * (1.0 / 6.0)
    tot = adv + tr
    tot_ref[...] = jnp.full((1, 1), tot, jnp.float32)
    adv_ref[...] = jnp.full((1, 1), adv, jnp.float32)
    tr_ref[...] = jnp.full((1, 1), tr, jnp.float32)


_finish = pl.pallas_call(
    _finish_body,
    out_shape=(jax.ShapeDtypeStruct((1, 1), jnp.float32),) * 3,
)


def kernel(fake_validity_voxel_0, fake_validity_voxel_1, fake_validity_program,
           voxel_feature, att, mask, program_target_ratio,
           pooled_program_feature_from_voxel, cross_edge_voxel_index,
           cross_edge_program_index, program_class_cluster,
           max_out_program_index, area_index_in_voxel_feature):
    area = jnp.take(voxel_feature, area_index_in_voxel_feature, axis=1)
    partials = _edge_accum(
        area,
        att.reshape(_NW * _NCH, _CH),
        cross_edge_voxel_index.astype(jnp.int32),
        cross_edge_program_index.astype(jnp.int32),
        program_class_cluster.astype(jnp.int32),
    )
    tot, adv, tr = _finish(
        partials,
        fake_validity_voxel_0.reshape(32, 128),
        fake_validity_voxel_1.reshape(32, 128),
        program_target_ratio.reshape(1, _NCLS),
    )
    total_loss = tot.reshape(())
    adversarial_loss = adv.reshape(())
    target_ratio_loss = tr.reshape(())
    link_prediction_loss = jnp.zeros(())
    return (total_loss, adversarial_loss, link_prediction_loss,
            target_ratio_loss)


# speedup vs baseline: 94.1385x; 1.1668x over previous
import functools

import jax
import jax.numpy as jnp
from jax import lax
from jax.experimental import pallas as pl
from jax.experimental.pallas import tpu as pltpu
from jax.experimental.pallas import tpu_sc as plsc

_NV = 100000
_E = 1600000
_NP = 2000
_NCLS = 6
_NW = 32

_EW = _E // _NW
_CH = 2000
_NCH = _EW // _CH
_NEV = _CH // 16

_mesh = plsc.VectorSubcoreMesh(core_axis_name="c", subcore_axis_name="s")
_sc_params = pltpu.CompilerParams(needs_layout_passes=False,
                                  use_tc_tiling_on_sc=False)


_UNROLL = 5


@functools.partial(
    pl.kernel,
    mesh=_mesh,
    out_type=jax.ShapeDtypeStruct((_NW, 96), jnp.float32),
    scratch_types=[
        pltpu.VMEM((_NV,), jnp.float32),
        pltpu.VMEM((_NP,), jnp.int32),
        pltpu.VMEM((2, _CH), jnp.float32),
        pltpu.VMEM((2, _CH), jnp.int32),
        pltpu.VMEM((2, _CH), jnp.int32),
        pltpu.VMEM((96,), jnp.float32),
        pltpu.SemaphoreType.DMA,
        pltpu.SemaphoreType.DMA,
        pltpu.SemaphoreType.DMA,
    ],
    compiler_params=_sc_params,
)
def _edge_accum(area_hbm, att_hbm, vox_hbm, prog_hbm, cls_hbm, out_hbm,
                area_t, cls_t, att_b, vox_b, prog_b, acc_b,
                sem_a, sem_v, sem_p):
    wid = lax.axis_index("s") * 2 + lax.axis_index("c")
    ebase = wid * _EW
    rbase = wid * _NCH
    lane = lax.iota(jnp.int32, 16)
    zero = jnp.zeros((16,), jnp.float32)

    def fire(c, slot):
        pltpu.async_copy(att_hbm.at[rbase + c], att_b.at[slot], sem_a)
        pltpu.async_copy(vox_hbm.at[pl.ds(ebase + c * _CH, _CH)],
                         vox_b.at[slot], sem_v)
        pltpu.async_copy(prog_hbm.at[pl.ds(ebase + c * _CH, _CH)],
                         prog_b.at[slot], sem_p)

    def drain(c, slot):
        pltpu.make_async_copy(att_hbm.at[rbase + c], att_b.at[slot],
                              sem_a).wait()
        pltpu.make_async_copy(vox_hbm.at[pl.ds(ebase + c * _CH, _CH)],
                              vox_b.at[slot], sem_v).wait()
        pltpu.make_async_copy(prog_hbm.at[pl.ds(ebase + c * _CH, _CH)],
                              prog_b.at[slot], sem_p).wait()

    fire(0, 0)
    pltpu.sync_copy(area_hbm, area_t)
    pltpu.sync_copy(cls_hbm, cls_t)
    for c6 in range(_NCLS):
        acc_b[pl.ds(c6 * 16, 16)] = zero

    def compute(slot):
        def vbody(u, carry):
            for k in range(_UNROLL):
                o = (u * _UNROLL + k) * 16
                vox = vox_b[slot, pl.ds(o, 16)]
                a = plsc.load_gather(area_t, [vox])
                val = att_b[slot, pl.ds(o, 16)] * a
                prog = prog_b[slot, pl.ds(o, 16)]
                cls = plsc.load_gather(cls_t, [prog])
                plsc.addupdate_scatter(acc_b, [cls * 16 + lane], val)
            return carry

        lax.fori_loop(0, _NEV // _UNROLL, vbody, 0)

    def outer(g, carry):
        c0 = g * 2
        fire(c0 + 1, 1)
        drain(c0, 0)
        compute(0)
        fire(c0 + 2, 0)
        drain(c0 + 1, 1)
        compute(1)
        return carry

    lax.fori_loop(0, (_NCH - 1) // 2, outer, 0)
    drain(_NCH - 1, 0)
    compute(0)
    pltpu.sync_copy(acc_b, out_hbm.at[wid])


def _finish_body(part_ref, fv0_ref, fv1_ref, tgt_ref, tot_ref, adv_ref, tr_ref):
    adv = (-(jnp.sum(fv0_ref[...]) * (1.0 / 4096.0))
           - (jnp.sum(fv1_ref[...]) * (1.0 / 4096.0)))
    x = part_ref[...]
    s = [jnp.sum(x[:, c * 16:(c + 1) * 16]) for c in range(_NCLS)]
    tot_w = s[0] + s[1] + s[2] + s[3] + s[4] + s[5]
    inv = 1.0 / (tot_w + 1e-16)
    t = tgt_ref[...]
    losses = []
    for c in range(_NCLS):
        d = s[c] * inv - jnp.sum(t[:, c:c + 1])
        ad = jnp.abs(d)
        losses.append(jnp.where(ad < 1.0, 0.5 * d * d, ad - 0.5))
    tr = (losses[0] + losses[1] + losses[2] + losses[3] + losses[4]
          + losses[5]) * (1.0 / 6.0)
    tot = adv + tr
    tot_ref[...] = jnp.full((1, 1), tot, jnp.float32)
    adv_ref[...] = jnp.full((1, 1), adv, jnp.float32)
    tr_ref[...] = jnp.full((1, 1), tr, jnp.float32)


_finish = pl.pallas_call(
    _finish_body,
    out_shape=(jax.ShapeDtypeStruct((1, 1), jnp.float32),) * 3,
)


def kernel(fake_validity_voxel_0, fake_validity_voxel_1, fake_validity_program,
           voxel_feature, att, mask, program_target_ratio,
           pooled_program_feature_from_voxel, cross_edge_voxel_index,
           cross_edge_program_index, program_class_cluster,
           max_out_program_index, area_index_in_voxel_feature):
    area = jnp.take(voxel_feature, area_index_in_voxel_feature, axis=1)
    partials = _edge_accum(
        area,
        att.reshape(_NW * _NCH, _CH),
        cross_edge_voxel_index.astype(jnp.int32),
        cross_edge_program_index.astype(jnp.int32),
        program_class_cluster.astype(jnp.int32),
    )
    tot, adv, tr = _finish(
        partials,
        fake_validity_voxel_0.reshape(32, 128),
        fake_validity_voxel_1.reshape(32, 128),
        program_target_ratio.reshape(1, _NCLS),
    )
    total_loss = tot.reshape(())
    adversarial_loss = adv.reshape(())
    target_ratio_loss = tr.reshape(())
    link_prediction_loss = jnp.zeros(())
    return (total_loss, adversarial_loss, link_prediction_loss,
            target_ratio_loss)
